# Initial kernel scaffold; baseline (speedup 1.0000x reference)
import jax
import jax.numpy as jnp
from jax.experimental import pallas as pl


def kernel(inputs, table):
    raise NotImplementedError("write your pallas kernel here")


# speedup vs baseline: 1.9254x; 1.9254x over previous
import functools

import jax
import jax.numpy as jnp
from jax import lax
from jax.experimental import pallas as pl
from jax.experimental.pallas import tpu as pltpu
from jax.experimental.pallas import tpu_sc as plsc

ROWS, COLS = 16384, 200
EMBED = 128
N_TOTAL = ROWS * COLS
NC, NS = 2, 16
NW = NC * NS
N_PER_W = N_TOTAL // NW
CHUNK = 128
N_CHUNKS = N_PER_W // CHUNK

_mesh = plsc.VectorSubcoreMesh(core_axis_name="c", subcore_axis_name="s")


@functools.partial(
    pl.kernel,
    out_type=jax.ShapeDtypeStruct((N_TOTAL, EMBED), jnp.float32),
    mesh=_mesh,
    scratch_types=[
        pltpu.VMEM((CHUNK,), jnp.int32),
        pltpu.VMEM((CHUNK, EMBED), jnp.float32),
        pltpu.SemaphoreType.DMA,
    ],
)
def _embed_sc(idx_hbm, table_hbm, out_hbm, idx_v, rows_v, sem):
    wid = lax.axis_index("s") * NC + lax.axis_index("c")
    base = wid * N_PER_W

    def body(g, carry):
        off = base + g * CHUNK
        pltpu.sync_copy(idx_hbm.at[pl.ds(off, CHUNK)], idx_v)
        pltpu.async_copy(table_hbm.at[idx_v], rows_v, sem).wait()
        pltpu.sync_copy(rows_v, out_hbm.at[pl.ds(off, CHUNK)])
        return carry

    lax.fori_loop(0, N_CHUNKS, body, 0)


def kernel(inputs, table):
    idx = inputs.reshape(-1).astype(jnp.int32)
    out = _embed_sc(idx, table)
    return out.reshape(ROWS, COLS, EMBED)


# speedup vs baseline: 16.3543x; 8.4941x over previous
import functools

import jax
import jax.numpy as jnp
from jax import lax
from jax.experimental import pallas as pl
from jax.experimental.pallas import tpu as pltpu
from jax.experimental.pallas import tpu_sc as plsc

ROWS, COLS = 16384, 200
VOCAB = 30
EMBED = 128
N_TOTAL = ROWS * COLS
NC, NS = 2, 16
NW = NC * NS
N_PER_W = N_TOTAL // NW
CHUNK = 128
GPB = 2
SUP = GPB * CHUNK
IB = 16
N_CHUNKS = N_PER_W // CHUNK
N_IB = N_CHUNKS // IB
SUPS_PER_IB = IB // GPB

_mesh = plsc.VectorSubcoreMesh(core_axis_name="c", subcore_axis_name="s")


@functools.partial(
    pl.kernel,
    out_type=jax.ShapeDtypeStruct((N_TOTAL, EMBED), jnp.float32),
    mesh=_mesh,
    scratch_types=[
        pltpu.VMEM_SHARED((VOCAB, EMBED), jnp.float32),
        pltpu.VMEM((IB, CHUNK), jnp.int32),
        pltpu.VMEM((2, SUP, EMBED), jnp.float32),
        pltpu.SemaphoreType.DMA,
        pltpu.SemaphoreType.DMA,
        pltpu.SemaphoreType.DMA,
        pltpu.SemaphoreType.DMA,
    ],
)
def _embed_sc(idx_hbm, table_hbm, out_hbm, table_sp, idx_v, rows_v,
              gsem0, gsem1, ssem0, ssem1):
    cid = lax.axis_index("c")
    sid = lax.axis_index("s")
    wid = sid * NC + cid
    chunk_base = wid * N_CHUNKS
    gsem = (gsem0, gsem1)
    ssem = (ssem0, ssem1)

    @pl.when(sid == 0)
    def _():
        pltpu.sync_copy(table_hbm, table_sp)

    plsc.subcore_barrier()

    def blk_body(ib, carry):
        pltpu.sync_copy(idx_hbm.at[pl.ds(chunk_base + ib * IB, IB)], idx_v)
        out_base = (chunk_base + ib * IB) * CHUNK

        g_descs = [None, None]
        s_descs = [None, None]
        for s in range(SUPS_PER_IB):
            b = s % 2
            if s_descs[b] is not None:
                s_descs[b].wait()
            g_descs[b] = [
                pltpu.async_copy(
                    table_sp.at[idx_v.at[s * GPB + g]],
                    rows_v.at[b, pl.ds(g * CHUNK, CHUNK)], gsem[b])
                for g in range(GPB)
            ]
            if s > 0:
                pb = 1 - b
                for d in g_descs[pb]:
                    d.wait()
                s_descs[pb] = pltpu.async_copy(
                    rows_v.at[pb],
                    out_hbm.at[pl.ds(out_base + (s - 1) * SUP, SUP)],
                    ssem[pb])
        lb = (SUPS_PER_IB - 1) % 2
        for d in g_descs[lb]:
            d.wait()
        s_descs[lb] = pltpu.async_copy(
            rows_v.at[lb],
            out_hbm.at[pl.ds(out_base + (SUPS_PER_IB - 1) * SUP, SUP)],
            ssem[lb])
        s_descs[0].wait()
        s_descs[1].wait()
        return carry

    lax.fori_loop(0, N_IB, blk_body, 0)


def kernel(inputs, table):
    idx = inputs.reshape(N_TOTAL // CHUNK, CHUNK).astype(jnp.int32)
    out = _embed_sc(idx, table)
    return out.reshape(ROWS, COLS, EMBED)


# speedup vs baseline: 18.0475x; 1.1035x over previous
import functools

import jax
import jax.numpy as jnp
from jax import lax
from jax.experimental import pallas as pl
from jax.experimental.pallas import tpu as pltpu
from jax.experimental.pallas import tpu_sc as plsc

ROWS, COLS = 16384, 200
VOCAB = 30
EMBED = 128
N_TOTAL = ROWS * COLS
NC, NS = 2, 16
NW = NC * NS
N_PER_W = N_TOTAL // NW
CHUNK = 128
GPB = 2
SUP = GPB * CHUNK
IB = 16
N_CHUNKS = N_PER_W // CHUNK
N_IB = N_CHUNKS // IB
SUPS_PER_IB = IB // GPB

_mesh = plsc.VectorSubcoreMesh(core_axis_name="c", subcore_axis_name="s")


@functools.partial(
    pl.kernel,
    out_type=jax.ShapeDtypeStruct((N_TOTAL, EMBED), jnp.float32),
    mesh=_mesh,
    scratch_types=[
        pltpu.VMEM_SHARED((VOCAB, EMBED), jnp.float32),
        pltpu.VMEM((2, IB, CHUNK), jnp.int32),
        pltpu.VMEM((2, SUP, EMBED), jnp.float32),
        pltpu.SemaphoreType.DMA,
        pltpu.SemaphoreType.DMA,
        pltpu.SemaphoreType.DMA,
        pltpu.SemaphoreType.DMA,
        pltpu.SemaphoreType.DMA,
    ],
)
def _embed_sc(idx_hbm, table_hbm, out_hbm, table_sp, idx_v, rows_v,
              isem, gsem0, gsem1, ssem0, ssem1):
    cid = lax.axis_index("c")
    sid = lax.axis_index("s")
    wid = sid * NC + cid
    chunk_base = wid * N_CHUNKS
    gsem = (gsem0, gsem1)
    ssem = (ssem0, ssem1)

    @pl.when(sid == 0)
    def _():
        pltpu.sync_copy(table_hbm, table_sp)

    plsc.subcore_barrier()

    def _wait_store(b):
        pltpu.make_async_copy(
            rows_v.at[b], out_hbm.at[pl.ds(chunk_base * CHUNK, SUP)],
            ssem[b]).wait()

    pltpu.async_copy(idx_hbm.at[pl.ds(chunk_base, IB)], idx_v.at[0], isem)

    def blk_body(ib, carry):
        cur = lax.rem(ib, 2)
        pltpu.make_async_copy(
            idx_hbm.at[pl.ds(chunk_base, IB)], idx_v.at[cur], isem).wait()

        out_base = (chunk_base + ib * IB) * CHUNK

        g_descs = [None, None]
        for s in range(SUPS_PER_IB):
            b = s % 2
            if s >= 2:
                _wait_store(b)
            else:
                @pl.when(ib >= 1)
                def _(b=b):
                    _wait_store(b)
            g_descs[b] = [
                pltpu.async_copy(
                    table_sp.at[idx_v.at[cur, s * GPB + g]],
                    rows_v.at[b, pl.ds(g * CHUNK, CHUNK)], gsem[b])
                for g in range(GPB)
            ]
            if s > 0:
                pb = 1 - b
                for d in g_descs[pb]:
                    d.wait()
                pltpu.async_copy(
                    rows_v.at[pb],
                    out_hbm.at[pl.ds(out_base + (s - 1) * SUP, SUP)],
                    ssem[pb])
        lb = (SUPS_PER_IB - 1) % 2
        for d in g_descs[lb]:
            d.wait()
        pltpu.async_copy(
            rows_v.at[lb],
            out_hbm.at[pl.ds(out_base + (SUPS_PER_IB - 1) * SUP, SUP)],
            ssem[lb])

        @pl.when(ib < N_IB - 1)
        def _():
            pltpu.async_copy(
                idx_hbm.at[pl.ds(chunk_base + (ib + 1) * IB, IB)],
                idx_v.at[1 - cur], isem)

        return carry

    lax.fori_loop(0, N_IB, blk_body, 0)

    _wait_store(0)
    _wait_store(1)


def kernel(inputs, table):
    idx = inputs.reshape(N_TOTAL // CHUNK, CHUNK).astype(jnp.int32)
    out = _embed_sc(idx, table)
    return out.reshape(ROWS, COLS, EMBED)


# speedup vs baseline: 18.5533x; 1.0280x over previous
import functools

import jax
import jax.numpy as jnp
from jax import lax
from jax.experimental import pallas as pl
from jax.experimental.pallas import tpu as pltpu
from jax.experimental.pallas import tpu_sc as plsc

ROWS, COLS = 16384, 200
VOCAB = 30
EMBED = 128
N_TOTAL = ROWS * COLS
NC, NS = 2, 16
NW = NC * NS
N_PER_W = N_TOTAL // NW
CHUNK = 128
NBUF = 4
IB = 16
N_CHUNKS = N_PER_W // CHUNK
N_IB = N_CHUNKS // IB

_mesh = plsc.VectorSubcoreMesh(core_axis_name="c", subcore_axis_name="s")


@functools.partial(
    pl.kernel,
    out_type=jax.ShapeDtypeStruct((N_TOTAL, EMBED), jnp.float32),
    mesh=_mesh,
    scratch_types=[
        pltpu.VMEM_SHARED((VOCAB, EMBED), jnp.float32),
        pltpu.VMEM((2, IB, CHUNK), jnp.int32),
        pltpu.VMEM((NBUF, CHUNK, EMBED), jnp.float32),
        pltpu.SemaphoreType.DMA,
        pltpu.SemaphoreType.DMA,
        pltpu.SemaphoreType.DMA,
        pltpu.SemaphoreType.DMA,
        pltpu.SemaphoreType.DMA,
        pltpu.SemaphoreType.DMA,
        pltpu.SemaphoreType.DMA,
        pltpu.SemaphoreType.DMA,
        pltpu.SemaphoreType.DMA,
    ],
)
def _embed_sc(idx_hbm, table_hbm, out_hbm, table_sp, idx_v, rows_v,
              isem, gsem0, gsem1, gsem2, gsem3, ssem0, ssem1, ssem2, ssem3):
    cid = lax.axis_index("c")
    sid = lax.axis_index("s")
    wid = sid * NC + cid
    chunk_base = wid * N_CHUNKS
    gsem = (gsem0, gsem1, gsem2, gsem3)
    ssem = (ssem0, ssem1, ssem2, ssem3)

    @pl.when(sid == 0)
    def _():
        pltpu.sync_copy(table_hbm, table_sp)

    plsc.subcore_barrier()

    def _wait_store(b):
        pltpu.make_async_copy(
            rows_v.at[b], out_hbm.at[pl.ds(chunk_base * CHUNK, CHUNK)],
            ssem[b]).wait()

    pltpu.async_copy(idx_hbm.at[pl.ds(chunk_base, IB)], idx_v.at[0], isem)

    def blk_body(ib, carry):
        cur = lax.rem(ib, 2)
        pltpu.make_async_copy(
            idx_hbm.at[pl.ds(chunk_base, IB)], idx_v.at[cur], isem).wait()

        out_base = (chunk_base + ib * IB) * CHUNK

        g_descs = [None] * NBUF
        for s in range(IB):
            b = s % NBUF
            if s >= NBUF:
                _wait_store(b)
            else:
                @pl.when(ib >= 1)
                def _(b=b):
                    _wait_store(b)
            g_descs[b] = pltpu.async_copy(
                table_sp.at[idx_v.at[cur, s]], rows_v.at[b], gsem[b])
            if s > 0:
                pb = (s - 1) % NBUF
                g_descs[pb].wait()
                pltpu.async_copy(
                    rows_v.at[pb],
                    out_hbm.at[pl.ds(out_base + (s - 1) * CHUNK, CHUNK)],
                    ssem[pb])
        lb = (IB - 1) % NBUF
        g_descs[lb].wait()
        pltpu.async_copy(
            rows_v.at[lb],
            out_hbm.at[pl.ds(out_base + (IB - 1) * CHUNK, CHUNK)],
            ssem[lb])

        @pl.when(ib < N_IB - 1)
        def _():
            pltpu.async_copy(
                idx_hbm.at[pl.ds(chunk_base + (ib + 1) * IB, IB)],
                idx_v.at[1 - cur], isem)

        return carry

    lax.fori_loop(0, N_IB, blk_body, 0)

    for b in range(NBUF):
        _wait_store(b)


def kernel(inputs, table):
    idx = inputs.reshape(N_TOTAL // CHUNK, CHUNK).astype(jnp.int32)
    out = _embed_sc(idx, table)
    return out.reshape(ROWS, COLS, EMBED)


# speedup vs baseline: 19.3247x; 1.0416x over previous
import functools

import jax
import jax.numpy as jnp
from jax import lax
from jax.experimental import pallas as pl
from jax.experimental.pallas import tpu as pltpu
from jax.experimental.pallas import tpu_sc as plsc

ROWS, COLS = 16384, 200
VOCAB = 30
EMBED = 128
N_TOTAL = ROWS * COLS
NC, NS = 2, 16
NW = NC * NS
R_PER_W = ROWS // NW
SPLIT = 128
NBUF = 4
IB = 16
N_IB = R_PER_W // IB

_mesh = plsc.VectorSubcoreMesh(core_axis_name="c", subcore_axis_name="s")


@functools.partial(
    pl.kernel,
    out_type=jax.ShapeDtypeStruct((N_TOTAL, EMBED), jnp.float32),
    mesh=_mesh,
    scratch_types=[
        pltpu.VMEM_SHARED((VOCAB, EMBED), jnp.float32),
        pltpu.VMEM((2, IB, COLS), jnp.int32),
        pltpu.VMEM((NBUF, COLS, EMBED), jnp.float32),
        pltpu.SemaphoreType.DMA,
        pltpu.SemaphoreType.DMA,
        pltpu.SemaphoreType.DMA,
        pltpu.SemaphoreType.DMA,
        pltpu.SemaphoreType.DMA,
        pltpu.SemaphoreType.DMA,
        pltpu.SemaphoreType.DMA,
        pltpu.SemaphoreType.DMA,
        pltpu.SemaphoreType.DMA,
    ],
)
def _embed_sc(idx_hbm, table_hbm, out_hbm, table_sp, idx_v, rows_v,
              isem, gsem0, gsem1, gsem2, gsem3, ssem0, ssem1, ssem2, ssem3):
    cid = lax.axis_index("c")
    sid = lax.axis_index("s")
    wid = sid * NC + cid
    row_base = wid * R_PER_W
    gsem = (gsem0, gsem1, gsem2, gsem3)
    ssem = (ssem0, ssem1, ssem2, ssem3)

    @pl.when(sid == 0)
    def _():
        pltpu.sync_copy(table_hbm, table_sp)

    plsc.subcore_barrier()

    def _wait_store(b):
        pltpu.make_async_copy(
            rows_v.at[b], out_hbm.at[pl.ds(row_base * COLS, COLS)],
            ssem[b]).wait()

    pltpu.async_copy(idx_hbm.at[pl.ds(row_base, IB)], idx_v.at[0], isem)

    def blk_body(ib, carry):
        cur = lax.rem(ib, 2)
        pltpu.make_async_copy(
            idx_hbm.at[pl.ds(row_base, IB)], idx_v.at[cur], isem).wait()

        out_base = (row_base + ib * IB) * COLS

        g_descs = [None] * NBUF
        for s in range(IB):
            b = s % NBUF
            if s >= NBUF:
                _wait_store(b)
            else:
                @pl.when(ib >= 1)
                def _(b=b):
                    _wait_store(b)
            g_descs[b] = [
                pltpu.async_copy(
                    table_sp.at[idx_v.at[cur, s, pl.ds(off, ln)]],
                    rows_v.at[b, pl.ds(off, ln)], gsem[b])
                for off, ln in ((0, SPLIT), (SPLIT, COLS - SPLIT))
            ]
            if s > 0:
                pb = (s - 1) % NBUF
                for d in g_descs[pb]:
                    d.wait()
                pltpu.async_copy(
                    rows_v.at[pb],
                    out_hbm.at[pl.ds(out_base + (s - 1) * COLS, COLS)],
                    ssem[pb])
        lb = (IB - 1) % NBUF
        for d in g_descs[lb]:
            d.wait()
        pltpu.async_copy(
            rows_v.at[lb],
            out_hbm.at[pl.ds(out_base + (IB - 1) * COLS, COLS)],
            ssem[lb])

        @pl.when(ib < N_IB - 1)
        def _():
            pltpu.async_copy(
                idx_hbm.at[pl.ds(row_base + (ib + 1) * IB, IB)],
                idx_v.at[1 - cur], isem)

        return carry

    lax.fori_loop(0, N_IB, blk_body, 0)

    for b in range(NBUF):
        _wait_store(b)


def kernel(inputs, table):
    out = _embed_sc(inputs.astype(jnp.int32), table)
    return out.reshape(ROWS, COLS, EMBED)
